# baseline (speedup 1.0000x reference)
import functools

import jax
import jax.numpy as jnp
from jax import lax
from jax.experimental import pallas as pl
from jax.experimental.pallas import tpu as pltpu
from jax.experimental.pallas import tpu_sc as plsc

NUM_CLASSES = 1000
HIDDEN = 128
BATCH = 16384

_INFO = plsc.get_sparse_core_info()
_NC, _NS = _INFO.num_cores, _INFO.num_subcores
_NW = _NC * _NS
_B_PER_W = BATCH // _NW
_IDX_MINOR = 128
_CHUNKS = _B_PER_W // _IDX_MINOR

_mesh = plsc.VectorSubcoreMesh(core_axis_name="c", subcore_axis_name="s")


@functools.partial(
    pl.kernel,
    mesh=_mesh,
    out_type=jax.ShapeDtypeStruct((BATCH, HIDDEN), jnp.float32),
    scratch_types=[
        pltpu.VMEM_SHARED((NUM_CLASSES + 1, HIDDEN), jnp.float32),
        pltpu.VMEM((_CHUNKS, _IDX_MINOR), jnp.int32),
        pltpu.VMEM((_B_PER_W, HIDDEN), jnp.float32),
        [pltpu.SemaphoreType.DMA] * _CHUNKS,
        pltpu.SemaphoreType.DMA,
    ],
)
def _embed_gather(labels_hbm, table_hbm, out_hbm, table_sp, idx_v, rows_v,
                  gsems, osem):
    sid = lax.axis_index("s")
    wid = sid * _NC + lax.axis_index("c")
    row0 = wid * _CHUNKS
    idx_cp = pltpu.async_copy(labels_hbm.at[pl.ds(row0, _CHUNKS)], idx_v, osem)
    @pl.when(sid == 0)
    def _stage_table():
        pltpu.sync_copy(table_hbm, table_sp)
    idx_cp.wait()
    plsc.subcore_barrier()
    gathers = []
    for j in range(_CHUNKS):
        gathers.append(
            pltpu.async_copy(
                table_sp.at[idx_v.at[j]],
                rows_v.at[pl.ds(j * _IDX_MINOR, _IDX_MINOR)],
                gsems[j],
            )
        )
    writes = []
    for j in range(_CHUNKS):
        gathers[j].wait()
        writes.append(
            pltpu.async_copy(
                rows_v.at[pl.ds(j * _IDX_MINOR, _IDX_MINOR)],
                out_hbm.at[pl.ds(wid * _B_PER_W + j * _IDX_MINOR, _IDX_MINOR)],
                osem,
            )
        )
    for c in writes:
        c.wait()


def kernel(labels, table):
    labels2d = labels.astype(jnp.int32).reshape(BATCH // _IDX_MINOR, _IDX_MINOR)
    return _embed_gather(labels2d, table)
